# baseline (speedup 1.0000x reference)
import numpy as np
import jax
import jax.numpy as jnp
from jax.experimental import pallas as pl
from jax.experimental.pallas import tpu as pltpu

_CHANNEL_IN = 256
_CHANNEL_OUT = 32768
_GROUP = 8
_BATCH = 128
_N_BLK = 2048
_NC_BLK = _N_BLK // _GROUP

_E = np.repeat(np.eye(_NC_BLK, dtype=np.float32), _GROUP, axis=1)
_K = np.broadcast_to(
    np.tile(np.arange(_GROUP, dtype=np.float32), _NC_BLK), (8, _N_BLK)
).copy()


def _body(x_ref, w_ref, e_ref, k_ref, o_ref):
    yt = jax.lax.dot_general(
        w_ref[...], x_ref[...],
        dimension_numbers=(((0,), (1,)), ((), ())),
        preferred_element_type=jnp.float32,
    )
    n, b = yt.shape
    r = yt.reshape(n // _GROUP, _GROUP, b)
    m = jnp.max(r, axis=1, keepdims=True)
    iota = jax.lax.broadcasted_iota(jnp.int32, r.shape, 1)
    first = jnp.min(jnp.where(r >= m, iota, _GROUP), axis=1).astype(jnp.float32)
    rep = jax.lax.dot_general(
        first, e_ref[...],
        dimension_numbers=(((0,), (0,)), ((), ())),
        preferred_element_type=jnp.float32,
    )
    o_ref[...] = (rep == k_ref[0:1, :]).astype(jnp.float32)


@jax.jit
def kernel(x, W):
    return pl.pallas_call(
        _body,
        grid=(_CHANNEL_OUT // _N_BLK,),
        in_specs=[
            pl.BlockSpec((_BATCH, _CHANNEL_IN), lambda i: (0, 0)),
            pl.BlockSpec((_CHANNEL_IN, _N_BLK), lambda i: (0, i)),
            pl.BlockSpec((_NC_BLK, _N_BLK), lambda i: (0, 0)),
            pl.BlockSpec((8, _N_BLK), lambda i: (0, 0)),
        ],
        out_specs=pl.BlockSpec((_BATCH, _N_BLK), lambda i: (0, i)),
        out_shape=jax.ShapeDtypeStruct((_BATCH, _CHANNEL_OUT), jnp.float32),
        compiler_params=pltpu.CompilerParams(
            dimension_semantics=("parallel",),
        ),
    )(x, W, jnp.asarray(_E), jnp.asarray(_K))


# speedup vs baseline: 1.0336x; 1.0336x over previous
import numpy as np
import jax
import jax.numpy as jnp
from jax.experimental import pallas as pl
from jax.experimental.pallas import tpu as pltpu

_CHANNEL_IN = 256
_CHANNEL_OUT = 32768
_GROUP = 8
_BATCH = 128
_N_BLK = 2048
_NC_BLK = _N_BLK // _GROUP

_E = np.repeat(np.eye(_NC_BLK, dtype=np.float32), _GROUP, axis=1)
_K = np.broadcast_to(
    np.tile(np.arange(_GROUP, dtype=np.float32), _NC_BLK), (8, _N_BLK)
).copy()
_IOTA8 = np.broadcast_to(
    np.arange(_GROUP, dtype=np.float32)[:, None], (_GROUP, 128)
).copy()


def _body(x_ref, w_ref, e_ref, k_ref, i8_ref, o_ref):
    yt = jax.lax.dot_general(
        w_ref[...], x_ref[...],
        dimension_numbers=(((0,), (1,)), ((), ())),
        preferred_element_type=jnp.float32,
    )
    n, b = yt.shape
    r = yt.reshape(n // _GROUP, _GROUP, b)
    m = jnp.max(r, axis=1, keepdims=True)
    iota = i8_ref[...].reshape(1, _GROUP, b)
    first = jnp.min(jnp.where(r >= m, iota, float(_GROUP)), axis=1)
    rep = jax.lax.dot_general(
        first.astype(jnp.bfloat16), e_ref[...],
        dimension_numbers=(((0,), (0,)), ((), ())),
        preferred_element_type=jnp.float32,
    )
    o_ref[...] = (rep == k_ref[0:1, :]).astype(jnp.float32)


@jax.jit
def kernel(x, W):
    return pl.pallas_call(
        _body,
        grid=(_CHANNEL_OUT // _N_BLK,),
        in_specs=[
            pl.BlockSpec((_BATCH, _CHANNEL_IN), lambda i: (0, 0)),
            pl.BlockSpec((_CHANNEL_IN, _N_BLK), lambda i: (0, i)),
            pl.BlockSpec((_NC_BLK, _N_BLK), lambda i: (0, 0)),
            pl.BlockSpec((8, _N_BLK), lambda i: (0, 0)),
            pl.BlockSpec((_GROUP, 128), lambda i: (0, 0)),
        ],
        out_specs=pl.BlockSpec((_BATCH, _N_BLK), lambda i: (0, i)),
        out_shape=jax.ShapeDtypeStruct((_BATCH, _CHANNEL_OUT), jnp.float32),
        compiler_params=pltpu.CompilerParams(
            dimension_semantics=("parallel",),
        ),
    )(x, W, jnp.asarray(_E, dtype=jnp.bfloat16), jnp.asarray(_K), jnp.asarray(_IOTA8))


# speedup vs baseline: 1.2990x; 1.2568x over previous
import jax
import jax.numpy as jnp
from jax.experimental import pallas as pl
from jax.experimental.pallas import tpu as pltpu

_CHANNEL_IN = 256
_CHANNEL_OUT = 32768
_GROUP = 8
_BATCH = 128
_N_BLK = 4096


def _body(x_ref, w_ref, o_ref):
    yt = jax.lax.dot_general(
        w_ref[...], x_ref[...],
        dimension_numbers=(((0,), (1,)), ((), ())),
        preferred_element_type=jnp.float32,
    )
    n, b = yt.shape
    r = yt.reshape(n // _GROUP, _GROUP, b)
    m = jnp.max(r, axis=1, keepdims=True)
    iota = jax.lax.broadcasted_iota(jnp.int32, r.shape, 1)
    first = jnp.min(jnp.where(r >= m, iota, _GROUP), axis=1, keepdims=True)
    oh = (iota == first).astype(jnp.float32).reshape(n, b)
    o_ref[...] = oh.T


@jax.jit
def kernel(x, W):
    return pl.pallas_call(
        _body,
        grid=(_CHANNEL_OUT // _N_BLK,),
        in_specs=[
            pl.BlockSpec((_BATCH, _CHANNEL_IN), lambda i: (0, 0)),
            pl.BlockSpec((_CHANNEL_IN, _N_BLK), lambda i: (0, i)),
        ],
        out_specs=pl.BlockSpec((_BATCH, _N_BLK), lambda i: (0, i)),
        out_shape=jax.ShapeDtypeStruct((_BATCH, _CHANNEL_OUT), jnp.float32),
        compiler_params=pltpu.CompilerParams(
            dimension_semantics=("parallel",),
        ),
    )(x, W)


# speedup vs baseline: 1.3369x; 1.0292x over previous
import jax
import jax.numpy as jnp
from jax.experimental import pallas as pl
from jax.experimental.pallas import tpu as pltpu

_CHANNEL_IN = 256
_CHANNEL_OUT = 32768
_GROUP = 8
_BATCH = 128
_N_BLK = 8192


def _body(x_ref, w_ref, o_ref):
    yt = jax.lax.dot_general(
        w_ref[...], x_ref[...],
        dimension_numbers=(((0,), (1,)), ((), ())),
        preferred_element_type=jnp.float32,
    )
    n, b = yt.shape
    r = yt.reshape(n // _GROUP, _GROUP, b)
    m = jnp.max(r, axis=1, keepdims=True)
    iota = jax.lax.broadcasted_iota(jnp.int32, r.shape, 1)
    first = jnp.min(jnp.where(r >= m, iota, _GROUP), axis=1, keepdims=True)
    oh = (iota == first).astype(jnp.float32).reshape(n, b)
    o_ref[...] = oh.T


@jax.jit
def kernel(x, W):
    return pl.pallas_call(
        _body,
        grid=(_CHANNEL_OUT // _N_BLK,),
        in_specs=[
            pl.BlockSpec((_BATCH, _CHANNEL_IN), lambda i: (0, 0)),
            pl.BlockSpec((_CHANNEL_IN, _N_BLK), lambda i: (0, i)),
        ],
        out_specs=pl.BlockSpec((_BATCH, _N_BLK), lambda i: (0, i)),
        out_shape=jax.ShapeDtypeStruct((_BATCH, _CHANNEL_OUT), jnp.float32),
        compiler_params=pltpu.CompilerParams(
            dimension_semantics=("parallel",),
        ),
    )(x, W)
